# baseline (speedup 1.0000x reference)
import functools

import jax
import jax.numpy as jnp
from jax import lax
from jax.experimental import pallas as pl
from jax.experimental.pallas import tpu as pltpu
from jax.experimental.pallas import tpu_sc as plsc

_E = 32
_D = 512
_H = 128
_N = 16384
_T = 128
_NH = _N // 2
_PH = _NH + _E * _T
_P = 2 * _PH
_NT = _P // _T
_NTH = _NT // 2

_NS = 16
_CT = _NH // _NS
_OH = _PH // _NS
_GC = 64
_NGC = _OH // _GC


def _router_body(x_ref, w_ref, b_ref, o_ref):
    logits = lax.dot_general(x_ref[...], w_ref[...], (((1,), (1,)), ((), ())),
                             preferred_element_type=jnp.float32)
    logits = logits + b_ref[...]
    o_ref[...] = jnp.argmax(logits, axis=1).astype(jnp.int32)[:, None]


def _router(xf, router_w, router_b):
    rows = 512
    out = pl.pallas_call(
        _router_body,
        grid=(_N // rows,),
        in_specs=[
            pl.BlockSpec((rows, _D), lambda i: (i, 0)),
            pl.BlockSpec((_E, _D), lambda i: (0, 0)),
            pl.BlockSpec((1, _E), lambda i: (0, 0)),
        ],
        out_specs=pl.BlockSpec((rows, 1), lambda i: (i, 0)),
        out_shape=jax.ShapeDtypeStruct((_N, 1), jnp.int32),
    )(xf, router_w, router_b.reshape(1, _E))
    return out.reshape(_N)


def _dispatch(xf, ridx):
    mesh = plsc.VectorSubcoreMesh(core_axis_name="c", subcore_axis_name="s")

    @functools.partial(
        pl.kernel,
        mesh=mesh,
        out_type=(
            jax.ShapeDtypeStruct((_P, _D), jnp.float32),
            jax.ShapeDtypeStruct((_N,), jnp.int32),
            jax.ShapeDtypeStruct((_NT,), jnp.int32),
        ),
        scratch_types=[
            pltpu.VMEM((_CT + 16,), jnp.int32),
            pltpu.VMEM((_E,), jnp.int32),
            pltpu.VMEM((_NS * _E,), jnp.int32),
            pltpu.VMEM((_CT + 16,), jnp.int32),
            pltpu.VMEM((_CT // _GC, _GC), jnp.int32),
            pltpu.VMEM((_CT,), jnp.int32),
            pltpu.VMEM((_CT // _GC, _GC), jnp.int32),
            pltpu.VMEM((_OH,), jnp.int32),
            pltpu.VMEM((2, _GC, _D), jnp.float32),
            pltpu.VMEM((_NTH,), jnp.int32),
            pltpu.VMEM((80,), jnp.int32),
            pltpu.VMEM((_NGC, _GC), jnp.int32),
            pltpu.SMEM((_E,), jnp.int32),
            pltpu.SMEM((_E,), jnp.int32),
            pltpu.VMEM_SHARED((_NS * _E,), jnp.int32),
            pltpu.VMEM_SHARED((_PH,), jnp.int32),
            pltpu.SemaphoreType.DMA,
            pltpu.SemaphoreType.DMA,
        ],
    )
    def k(x_hbm, ridx_hbm, xs_hbm, pos_hbm, te_hbm,
          ridx_v, cnt_v, tbl_v, posl_v, posl2_v, posg_v, tok_v, prm_v, buf_v,
          te_v, tp_v, gidx_v, cnt_s, off_s, tbl_sp, perm_sp, sem0, sem1):
        c = lax.axis_index("c")
        s = lax.axis_index("s")
        lanes = lax.broadcasted_iota(jnp.int32, (16,), 0)
        tok_base = c * _NH + s * _CT
        half_base = c * _PH

        pltpu.sync_copy(ridx_hbm.at[pl.ds(tok_base, _CT)], ridx_v.at[pl.ds(0, _CT)])

        def zero_body(e, _):
            cnt_s[e] = 0
            return 0
        lax.fori_loop(0, _E, zero_body, 0)

        def hist_body(i, _):
            e = ridx_v[pl.ds(i, 16)][0]
            cnt_s[e] = cnt_s[e] + 1
            return 0
        lax.fori_loop(0, _CT, hist_body, 0)

        for j in range(_OH // 16):
            prm_v[pl.ds(j * 16, 16)] = jnp.zeros((16,), jnp.int32)
        pltpu.sync_copy(prm_v, perm_sp.at[pl.ds(s * _OH, _OH)])

        for j in range(2):
            v = jnp.zeros((16,), jnp.int32)
            for l in range(16):
                v = jnp.where(lanes == l, cnt_s[j * 16 + l], v)
            cnt_v[pl.ds(j * 16, 16)] = v
        pltpu.sync_copy(cnt_v, tbl_sp.at[pl.ds(s * _E, _E)])
        plsc.subcore_barrier()

        pltpu.sync_copy(tbl_sp, tbl_v)
        tot0 = jnp.zeros((16,), jnp.int32)
        tot1 = jnp.zeros((16,), jnp.int32)
        pre0 = jnp.zeros((16,), jnp.int32)
        pre1 = jnp.zeros((16,), jnp.int32)
        for w in range(_NS):
            r0 = tbl_v[pl.ds(w * _E, 16)]
            r1 = tbl_v[pl.ds(w * _E + 16, 16)]
            tot0 = tot0 + r0
            tot1 = tot1 + r1
            keep = jnp.where(w < s, 1, 0)
            pre0 = pre0 + r0 * keep
            pre1 = pre1 + r1 * keep
        tp_v[pl.ds(0, 16)] = tot0
        tp_v[pl.ds(16, 16)] = tot1
        tp_v[pl.ds(32, 16)] = pre0
        tp_v[pl.ds(48, 16)] = pre1
        tp_v[pl.ds(64, 16)] = jnp.zeros((16,), jnp.int32)
        tile_starts = []
        run = jnp.int32(0)
        for e in range(_E):
            tot = tp_v[pl.ds(e, 16)][0]
            pre = tp_v[pl.ds(32 + e, 16)][0]
            off_s[e] = run + pre
            tile_starts.append(run // _T)
            run = run + ((tot + (_T - 1)) & ~(_T - 1))
        total_tiles = run // _T

        @pl.when(s == 0)
        def _():
            for j in range(_NTH // 16):
                kvec = j * 16 + lanes
                acc = jnp.zeros((16,), jnp.int32)
                for ts in tile_starts:
                    acc = acc + jnp.where(kvec >= ts, 1, 0)
                te_v[pl.ds(j * 16, 16)] = acc - 1
            pltpu.sync_copy(te_v, te_hbm.at[pl.ds(c * _NTH, _NTH)])

        def pos_body(i, _):
            e = ridx_v[pl.ds(i, 16)][0]
            p = off_s[e]
            off_s[e] = p + 1
            win = posl_v[pl.ds(i, 16)]
            posl_v[pl.ds(i, 16)] = jnp.where(lanes == 0, p, win)
            return 0
        lax.fori_loop(0, _CT, pos_body, 0)

        for j in range(_CT // _GC):
            for q in range(_GC // 16):
                pv = posl_v[pl.ds(j * _GC + q * 16, 16)]
                posl2_v[j, pl.ds(q * 16, 16)] = pv
                posg_v[pl.ds(j * _GC + q * 16, 16)] = half_base + pv
                tok_v[j, pl.ds(q * 16, 16)] = tok_base + j * _GC + q * 16 + lanes + 1
        for j in range(_CT // _GC):
            pltpu.sync_copy(tok_v.at[j], perm_sp.at[posl2_v.at[j]], add=True)
        pltpu.sync_copy(posg_v, pos_hbm.at[pl.ds(tok_base, _CT)])
        plsc.subcore_barrier()

        pltpu.sync_copy(perm_sp.at[pl.ds(s * _OH, _OH)], prm_v)
        for j in range(_NGC):
            for q in range(_GC // 16):
                v = prm_v[pl.ds(j * _GC + q * 16, 16)]
                fallback = (half_base + s * _OH + j * _GC + q * 16 + lanes) & (_N - 1)
                gidx_v[j, pl.ds(q * 16, 16)] = jnp.where(v == 0, fallback, v - 1)
        row_base = half_base + s * _OH
        sems = (sem0, sem1)
        cps = [None, None]
        cps[0] = pltpu.async_copy(x_hbm.at[gidx_v.at[0]], buf_v.at[0], sem0)
        cps[1] = pltpu.async_copy(x_hbm.at[gidx_v.at[1]], buf_v.at[1], sem1)
        for j in range(_NGC):
            cps[j % 2].wait()
            pltpu.sync_copy(buf_v.at[j % 2],
                            xs_hbm.at[pl.ds(row_base + j * _GC, _GC)])
            if j + 2 < _NGC:
                cps[j % 2] = pltpu.async_copy(
                    x_hbm.at[gidx_v.at[j + 2]], buf_v.at[j % 2], sems[j % 2])

    return k(xf, ridx)


_NW = 32
_CGC = 64
_CNC = _N // (_NW * _CGC)


def _combine(out_sorted, pos):
    idx3 = pos.reshape(_NW, _CNC, _CGC)
    mesh = plsc.VectorSubcoreMesh(core_axis_name="c", subcore_axis_name="s")

    @functools.partial(
        pl.kernel,
        mesh=mesh,
        out_type=jax.ShapeDtypeStruct((_N, _D), jnp.float32),
        scratch_types=[
            pltpu.VMEM((_CNC, _CGC), jnp.int32),
            pltpu.VMEM((2, _CGC, _D), jnp.float32),
            pltpu.SemaphoreType.DMA,
            pltpu.SemaphoreType.DMA,
        ],
    )
    def k(src_hbm, idx_hbm, out_hbm, idx_v, buf_v, sem0, sem1):
        wid = lax.axis_index("s") * 2 + lax.axis_index("c")
        base = wid * (_CNC * _CGC)
        pltpu.sync_copy(idx_hbm.at[wid], idx_v)
        sems = (sem0, sem1)
        cps = [None, None]
        cps[0] = pltpu.async_copy(src_hbm.at[idx_v.at[0]], buf_v.at[0], sem0)
        cps[1] = pltpu.async_copy(src_hbm.at[idx_v.at[1]], buf_v.at[1], sem1)
        for j in range(_CNC):
            cps[j % 2].wait()
            pltpu.sync_copy(buf_v.at[j % 2], out_hbm.at[pl.ds(base + j * _CGC, _CGC)])
            if j + 2 < _CNC:
                cps[j % 2] = pltpu.async_copy(
                    src_hbm.at[idx_v.at[j + 2]], buf_v.at[j % 2], sems[j % 2])

    return k(out_sorted, idx3)


def _moe_tile_body(s_ref, x_ref, w1_ref, b1_ref, w2_ref, b2_ref, o_ref):
    h = jnp.dot(x_ref[...], w1_ref[0], preferred_element_type=jnp.float32)
    h = jnp.maximum(h + b1_ref[0, 0], 0.0)
    y = jnp.dot(h, w2_ref[0], preferred_element_type=jnp.float32)
    o_ref[...] = jnp.maximum(y + b2_ref[0, 0], 0.0)


def _grouped_mlp(tile_expert, x_sorted, w1, b1, w2, b2):
    grid_spec = pltpu.PrefetchScalarGridSpec(
        num_scalar_prefetch=1,
        grid=(_NT,),
        in_specs=[
            pl.BlockSpec((_T, _D), lambda i, s: (i, 0)),
            pl.BlockSpec((1, _D, _H), lambda i, s: (s[i], 0, 0)),
            pl.BlockSpec((1, 1, _H), lambda i, s: (s[i], 0, 0)),
            pl.BlockSpec((1, _H, _D), lambda i, s: (s[i], 0, 0)),
            pl.BlockSpec((1, 1, _D), lambda i, s: (s[i], 0, 0)),
        ],
        out_specs=pl.BlockSpec((_T, _D), lambda i, s: (i, 0)),
    )
    return pl.pallas_call(
        _moe_tile_body,
        grid_spec=grid_spec,
        out_shape=jax.ShapeDtypeStruct((_P, _D), jnp.float32),
    )(tile_expert, x_sorted, w1, b1.reshape(_E, 1, _H), w2, b2.reshape(_E, 1, _D))


def kernel(x, router_w, router_b, weights1, bias1, weights2, bias2):
    Bb, Kk, D = x.shape
    xf = x.reshape(-1, D)
    ridx = _router(xf, router_w, router_b)
    x_sorted, pos, tile_expert = _dispatch(xf, ridx)
    out_sorted = _grouped_mlp(tile_expert, x_sorted, weights1, bias1, weights2, bias2)
    out = _combine(out_sorted, pos)
    return out.reshape(Bb, Kk, D)


# speedup vs baseline: 1.0342x; 1.0342x over previous
import functools

import jax
import jax.numpy as jnp
from jax import lax
from jax.experimental import pallas as pl
from jax.experimental.pallas import tpu as pltpu
from jax.experimental.pallas import tpu_sc as plsc

_E = 32
_D = 512
_H = 128
_N = 16384
_T = 256
_NH = _N // 2
_PH = _NH + _E * _T
_P = 2 * _PH
_NT = _P // _T
_NTH = _NT // 2

_NS = 16
_CT = _NH // _NS
_OH = _PH // _NS
_GC = 64
_NGC = _OH // _GC


def _router_body(x_ref, w_ref, b_ref, o_ref):
    logits = lax.dot_general(x_ref[...], w_ref[...], (((1,), (1,)), ((), ())),
                             preferred_element_type=jnp.float32)
    logits = logits + b_ref[...]
    o_ref[...] = jnp.argmax(logits, axis=1).astype(jnp.int32)[:, None]


def _router(xf, router_w, router_b):
    rows = 512
    out = pl.pallas_call(
        _router_body,
        grid=(_N // rows,),
        in_specs=[
            pl.BlockSpec((rows, _D), lambda i: (i, 0)),
            pl.BlockSpec((_E, _D), lambda i: (0, 0)),
            pl.BlockSpec((1, _E), lambda i: (0, 0)),
        ],
        out_specs=pl.BlockSpec((rows, 1), lambda i: (i, 0)),
        out_shape=jax.ShapeDtypeStruct((_N, 1), jnp.int32),
    )(xf, router_w, router_b.reshape(1, _E))
    return out.reshape(_N)


def _dispatch(xf, ridx):
    mesh = plsc.VectorSubcoreMesh(core_axis_name="c", subcore_axis_name="s")

    @functools.partial(
        pl.kernel,
        mesh=mesh,
        out_type=(
            jax.ShapeDtypeStruct((_P, _D), jnp.float32),
            jax.ShapeDtypeStruct((_N,), jnp.int32),
            jax.ShapeDtypeStruct((_NT,), jnp.int32),
        ),
        scratch_types=[
            pltpu.VMEM((_CT + 16,), jnp.int32),
            pltpu.VMEM((_E,), jnp.int32),
            pltpu.VMEM((_NS * _E,), jnp.int32),
            pltpu.VMEM((_CT + 16,), jnp.int32),
            pltpu.VMEM((_CT // _GC, _GC), jnp.int32),
            pltpu.VMEM((_CT,), jnp.int32),
            pltpu.VMEM((_CT // _GC, _GC), jnp.int32),
            pltpu.VMEM((_OH,), jnp.int32),
            pltpu.VMEM((2, _GC, _D), jnp.float32),
            pltpu.VMEM((_NTH,), jnp.int32),
            pltpu.VMEM((80,), jnp.int32),
            pltpu.VMEM((_NGC, _GC), jnp.int32),
            pltpu.SMEM((_E,), jnp.int32),
            pltpu.SMEM((_E,), jnp.int32),
            pltpu.VMEM_SHARED((_NS * _E,), jnp.int32),
            pltpu.VMEM_SHARED((_PH,), jnp.int32),
            pltpu.SemaphoreType.DMA,
            pltpu.SemaphoreType.DMA,
        ],
    )
    def k(x_hbm, ridx_hbm, xs_hbm, pos_hbm, te_hbm,
          ridx_v, cnt_v, tbl_v, posl_v, posl2_v, posg_v, tok_v, prm_v, buf_v,
          te_v, tp_v, gidx_v, cnt_s, off_s, tbl_sp, perm_sp, sem0, sem1):
        c = lax.axis_index("c")
        s = lax.axis_index("s")
        lanes = lax.broadcasted_iota(jnp.int32, (16,), 0)
        tok_base = c * _NH + s * _CT
        half_base = c * _PH

        pltpu.sync_copy(ridx_hbm.at[pl.ds(tok_base, _CT)], ridx_v.at[pl.ds(0, _CT)])

        def zero_body(e, _):
            cnt_s[e] = 0
            return 0
        lax.fori_loop(0, _E, zero_body, 0)

        def hist_body(i, _):
            e = ridx_v[pl.ds(i, 16)][0]
            cnt_s[e] = cnt_s[e] + 1
            return 0
        lax.fori_loop(0, _CT, hist_body, 0)

        for j in range(_OH // 16):
            prm_v[pl.ds(j * 16, 16)] = jnp.zeros((16,), jnp.int32)
        pltpu.sync_copy(prm_v, perm_sp.at[pl.ds(s * _OH, _OH)])

        for j in range(2):
            v = jnp.zeros((16,), jnp.int32)
            for l in range(16):
                v = jnp.where(lanes == l, cnt_s[j * 16 + l], v)
            cnt_v[pl.ds(j * 16, 16)] = v
        pltpu.sync_copy(cnt_v, tbl_sp.at[pl.ds(s * _E, _E)])
        plsc.subcore_barrier()

        pltpu.sync_copy(tbl_sp, tbl_v)
        tot0 = jnp.zeros((16,), jnp.int32)
        tot1 = jnp.zeros((16,), jnp.int32)
        pre0 = jnp.zeros((16,), jnp.int32)
        pre1 = jnp.zeros((16,), jnp.int32)
        for w in range(_NS):
            r0 = tbl_v[pl.ds(w * _E, 16)]
            r1 = tbl_v[pl.ds(w * _E + 16, 16)]
            tot0 = tot0 + r0
            tot1 = tot1 + r1
            keep = jnp.where(w < s, 1, 0)
            pre0 = pre0 + r0 * keep
            pre1 = pre1 + r1 * keep
        tp_v[pl.ds(0, 16)] = tot0
        tp_v[pl.ds(16, 16)] = tot1
        tp_v[pl.ds(32, 16)] = pre0
        tp_v[pl.ds(48, 16)] = pre1
        tp_v[pl.ds(64, 16)] = jnp.zeros((16,), jnp.int32)
        tile_starts = []
        run = jnp.int32(0)
        for e in range(_E):
            tot = tp_v[pl.ds(e, 16)][0]
            pre = tp_v[pl.ds(32 + e, 16)][0]
            off_s[e] = run + pre
            tile_starts.append(run // _T)
            run = run + ((tot + (_T - 1)) & ~(_T - 1))
        total_tiles = run // _T

        @pl.when(s == 0)
        def _():
            for j in range(_NTH // 16):
                kvec = j * 16 + lanes
                acc = jnp.zeros((16,), jnp.int32)
                for ts in tile_starts:
                    acc = acc + jnp.where(kvec >= ts, 1, 0)
                te_v[pl.ds(j * 16, 16)] = acc - 1
            pltpu.sync_copy(te_v, te_hbm.at[pl.ds(c * _NTH, _NTH)])

        def pos_body(i, _):
            e = ridx_v[pl.ds(i, 16)][0]
            p = off_s[e]
            off_s[e] = p + 1
            win = posl_v[pl.ds(i, 16)]
            posl_v[pl.ds(i, 16)] = jnp.where(lanes == 0, p, win)
            return 0
        lax.fori_loop(0, _CT, pos_body, 0)

        for j in range(_CT // _GC):
            for q in range(_GC // 16):
                pv = posl_v[pl.ds(j * _GC + q * 16, 16)]
                posl2_v[j, pl.ds(q * 16, 16)] = pv
                posg_v[pl.ds(j * _GC + q * 16, 16)] = half_base + pv
                tok_v[j, pl.ds(q * 16, 16)] = tok_base + j * _GC + q * 16 + lanes + 1
        for j in range(_CT // _GC):
            pltpu.sync_copy(tok_v.at[j], perm_sp.at[posl2_v.at[j]], add=True)
        pltpu.sync_copy(posg_v, pos_hbm.at[pl.ds(tok_base, _CT)])
        plsc.subcore_barrier()

        pltpu.sync_copy(perm_sp.at[pl.ds(s * _OH, _OH)], prm_v)
        for j in range(_NGC):
            for q in range(_GC // 16):
                v = prm_v[pl.ds(j * _GC + q * 16, 16)]
                fallback = (half_base + s * _OH + j * _GC + q * 16 + lanes) & (_N - 1)
                gidx_v[j, pl.ds(q * 16, 16)] = jnp.where(v == 0, fallback, v - 1)
        row_base = half_base + s * _OH
        sems = (sem0, sem1)
        cps = [None, None]
        cps[0] = pltpu.async_copy(x_hbm.at[gidx_v.at[0]], buf_v.at[0], sem0)
        cps[1] = pltpu.async_copy(x_hbm.at[gidx_v.at[1]], buf_v.at[1], sem1)
        for j in range(_NGC):
            cps[j % 2].wait()
            pltpu.sync_copy(buf_v.at[j % 2],
                            xs_hbm.at[pl.ds(row_base + j * _GC, _GC)])
            if j + 2 < _NGC:
                cps[j % 2] = pltpu.async_copy(
                    x_hbm.at[gidx_v.at[j + 2]], buf_v.at[j % 2], sems[j % 2])

    return k(xf, ridx)


_NW = 32
_CGC = 64
_CNC = _N // (_NW * _CGC)


def _combine(out_sorted, pos):
    idx3 = pos.reshape(_NW, _CNC, _CGC)
    mesh = plsc.VectorSubcoreMesh(core_axis_name="c", subcore_axis_name="s")

    @functools.partial(
        pl.kernel,
        mesh=mesh,
        out_type=jax.ShapeDtypeStruct((_N, _D), jnp.float32),
        scratch_types=[
            pltpu.VMEM((_CNC, _CGC), jnp.int32),
            pltpu.VMEM((2, _CGC, _D), jnp.float32),
            pltpu.SemaphoreType.DMA,
            pltpu.SemaphoreType.DMA,
        ],
    )
    def k(src_hbm, idx_hbm, out_hbm, idx_v, buf_v, sem0, sem1):
        wid = lax.axis_index("s") * 2 + lax.axis_index("c")
        base = wid * (_CNC * _CGC)
        pltpu.sync_copy(idx_hbm.at[wid], idx_v)
        sems = (sem0, sem1)
        cps = [None, None]
        cps[0] = pltpu.async_copy(src_hbm.at[idx_v.at[0]], buf_v.at[0], sem0)
        cps[1] = pltpu.async_copy(src_hbm.at[idx_v.at[1]], buf_v.at[1], sem1)
        for j in range(_CNC):
            cps[j % 2].wait()
            pltpu.sync_copy(buf_v.at[j % 2], out_hbm.at[pl.ds(base + j * _CGC, _CGC)])
            if j + 2 < _CNC:
                cps[j % 2] = pltpu.async_copy(
                    src_hbm.at[idx_v.at[j + 2]], buf_v.at[j % 2], sems[j % 2])

    return k(out_sorted, idx3)


def _moe_tile_body(s_ref, x_ref, w1_ref, b1_ref, w2_ref, b2_ref, o_ref):
    h = jnp.dot(x_ref[...], w1_ref[0], preferred_element_type=jnp.float32)
    h = jnp.maximum(h + b1_ref[0, 0], 0.0)
    y = jnp.dot(h, w2_ref[0], preferred_element_type=jnp.float32)
    o_ref[...] = jnp.maximum(y + b2_ref[0, 0], 0.0)


def _grouped_mlp(tile_expert, x_sorted, w1, b1, w2, b2):
    grid_spec = pltpu.PrefetchScalarGridSpec(
        num_scalar_prefetch=1,
        grid=(_NT,),
        in_specs=[
            pl.BlockSpec((_T, _D), lambda i, s: (i, 0)),
            pl.BlockSpec((1, _D, _H), lambda i, s: (s[i], 0, 0)),
            pl.BlockSpec((1, 1, _H), lambda i, s: (s[i], 0, 0)),
            pl.BlockSpec((1, _H, _D), lambda i, s: (s[i], 0, 0)),
            pl.BlockSpec((1, 1, _D), lambda i, s: (s[i], 0, 0)),
        ],
        out_specs=pl.BlockSpec((_T, _D), lambda i, s: (i, 0)),
    )
    return pl.pallas_call(
        _moe_tile_body,
        grid_spec=grid_spec,
        out_shape=jax.ShapeDtypeStruct((_P, _D), jnp.float32),
    )(tile_expert, x_sorted, w1, b1.reshape(_E, 1, _H), w2, b2.reshape(_E, 1, _D))


def kernel(x, router_w, router_b, weights1, bias1, weights2, bias2):
    Bb, Kk, D = x.shape
    xf = x.reshape(-1, D)
    ridx = _router(xf, router_w, router_b)
    x_sorted, pos, tile_expert = _dispatch(xf, ridx)
    out_sorted = _grouped_mlp(tile_expert, x_sorted, weights1, bias1, weights2, bias2)
    out = _combine(out_sorted, pos)
    return out.reshape(Bb, Kk, D)
